# baseline (speedup 1.0000x reference)
import functools

import jax
import jax.numpy as jnp
from jax import lax
from jax.experimental import pallas as pl
from jax.experimental.pallas import tpu as pltpu
from jax.experimental.pallas import tpu_sc as plsc

from jax._src import core as _jax_core
from jax._src.pallas.mosaic import sc_lowering as _scl
from jax.experimental.mosaic.dialects import tpu as _tpu_dialect

_lane_perm_p = _jax_core.Primitive("sc_lane_perm")


@_lane_perm_p.def_abstract_eval
def _lane_perm_abs(x, idx):
    return _jax_core.ShapedArray(x.shape, x.dtype)


@_scl.register_lowering_rule(_lane_perm_p)
def _lane_perm_lowering(ctx, x, idx):
    return _tpu_dialect.dynamic_gather(x, idx, dimensions=[0])


def _lane_perm(x, idx):
    return _lane_perm_p.bind(x, idx)


_VOCAB = 1000000
_DIM = 64
_B = 16384

_NC = 2
_NS = 16
_NW = _NC * _NS
_BPW = _B // _NW
_NGRP = _BPW // 16
_NBUF = 6


def _pack_sorted(sv):
    prevb = jnp.concatenate(
        [jnp.full((_NW, 1), -1, jnp.int32), sv[:, :-1] >> 7], axis=1)
    nb = ((sv >> 7) != prevb).astype(jnp.int32)
    uslot = (jnp.cumsum(nb, axis=1) - 1) % _NBUF
    return sv | (nb << 20) | (uslot << 21)


def _mk_ring(tbl_hbm, blk_v, sems):
    def fire(pv):
        vb = pl.multiple_of(((pv & 0xFFFFF) >> 7) * 128, 128)

        @pl.when(((pv >> 20) & 1) == 1)
        def _():
            pltpu.async_copy(
                tbl_hbm.at[pl.ds(0, _DIM), pl.ds(vb, 128)],
                blk_v.at[(pv >> 21) & 7],
                sems.at[(pv >> 21) & 7])

    def wait(pv):
        @pl.when(((pv >> 20) & 1) == 1)
        def _():
            pltpu.make_async_copy(
                tbl_hbm.at[pl.ds(0, _DIM), pl.ds(0, 128)],
                blk_v.at[(pv >> 21) & 7],
                sems.at[(pv >> 21) & 7]).wait()

    return fire, wait


def _tree16(vals, lane):
    for bit in (8, 4, 2, 1):
        vals = [jnp.where((lane & bit) == 0, vals[i], vals[i + bit])
                for i in range(bit)]
    return vals[0]


def _sc_crows_kernel(pc_hbm, win_hbm, crows_hbm, cidx_v, cblk_v, row_v,
                     csems):
    cid = lax.axis_index("c")
    sid = lax.axis_index("s")
    wid = cid * _NS + sid
    r = wid % 8
    pltpu.sync_copy(pc_hbm.at[pl.ds((wid // 8) * 8, 8)], cidx_v)
    lane = lax.iota(jnp.int32, 16)
    fire_c, wait_c = _mk_ring(win_hbm, cblk_v, csems)

    idxc0 = cidx_v[r, pl.ds(0, 16)]
    for j in range(_NBUF):
        fire_c(idxc0[j])

    def body(g, _):
        goff = g * 16
        noff = jnp.minimum(goff + 16, _BPW - 16)
        idxc = cidx_v[r, pl.ds(goff, 16)]
        idxc_n = cidx_v[r, pl.ds(noff, 16)]
        last = g == _NGRP - 1
        for j in range(16):
            pv = idxc[j]
            cslot = (pv >> 21) & 7
            wait_c(pv)
            v = pv & 0xFFFFF
            col = v & 127
            b16 = col & 112
            abc = lane * 0 + (col & 15)
            row = g * 8 + (j // 2)
            for k in range(_DIM // 16):
                vals = [
                    _lane_perm(cblk_v[cslot, 16 * k + l, pl.ds(b16, 16)],
                               abc)
                    for l in range(16)
                ]
                row_v[row, pl.ds((j % 2) * 64 + 16 * k, 16)] = (
                    _tree16(vals, lane))
            if j + _NBUF < 16:
                vn = idxc[j + _NBUF]
            else:
                vn = jnp.where(last, 0, idxc_n[j + _NBUF - 16])
            fire_c(vn)
        return 0

    lax.fori_loop(0, _NGRP, body, 0)
    pltpu.sync_copy(row_v, crows_hbm.at[pl.ds(wid * (_BPW // 2),
                                              _BPW // 2)])


@functools.partial(
    pl.kernel,
    out_type=jax.ShapeDtypeStruct((_B // 2, 128), jnp.float32),
    mesh=plsc.VectorSubcoreMesh(core_axis_name="c", subcore_axis_name="s"),
    compiler_params=pltpu.CompilerParams(use_tc_tiling_on_sc=True),
    scratch_types=[
        pltpu.VMEM((8, _BPW), jnp.int32),
        pltpu.VMEM((_NBUF, _DIM, 128), jnp.float32),
        pltpu.VMEM((_BPW // 2, 128), jnp.float32),
        pltpu.SemaphoreType.DMA((_NBUF,)),
    ],
)
def _sc_crows(pc_hbm, win_hbm, crows_hbm, cidx_v, cblk_v, row_v, csems):
    _sc_crows_kernel(pc_hbm, win_hbm, crows_hbm, cidx_v, cblk_v, row_v,
                     csems)


def _sc_simt_kernel(pt_hbm, m_hbm, wout_hbm, crows_hbm, sim_hbm,
                    tidx_v, m_v, tblk_v, crow_v, sim_v, sim_sh,
                    tsems, gsems):
    cid = lax.axis_index("c")
    sid = lax.axis_index("s")
    wid = cid * _NS + sid
    r = wid % 8
    slab = (wid // 8) * 8
    pltpu.sync_copy(pt_hbm.at[pl.ds(slab, 8)], tidx_v)
    pltpu.sync_copy(m_hbm.at[pl.ds(slab, 8)], m_v)
    lane = lax.iota(jnp.int32, 16)
    fire_t, wait_t = _mk_ring(wout_hbm, tblk_v, tsems)

    def fire_g(g, slot):
        mrow = m_v[r, pl.ds(g * 16, 16)] >> 1
        pltpu.async_copy(crows_hbm.at[mrow], crow_v.at[slot],
                         gsems.at[slot])

    def wait_g(slot):
        pltpu.make_async_copy(crows_hbm.at[pl.ds(0, 16)],
                              crow_v.at[slot], gsems.at[slot]).wait()

    idxt0 = tidx_v[r, pl.ds(0, 16)]
    for j in range(_NBUF):
        fire_t(idxt0[j])
    fire_g(0, 0)

    def body(g, _):
        goff = g * 16
        noff = jnp.minimum(goff + 16, _BPW - 16)
        idxt = tidx_v[r, pl.ds(goff, 16)]
        idxt_n = tidx_v[r, pl.ds(noff, 16)]
        mvec = m_v[r, pl.ds(goff, 16)]
        last = g == _NGRP - 1
        gslot = g & 1
        wait_g(gslot)

        @pl.when(jnp.logical_not(last))
        def _():
            fire_g(g + 1, 1 - gslot)

        vec = jnp.zeros((16,), jnp.float32)
        for j in range(16):
            pv = idxt[j]
            wait_t(pv)
            tslot = (pv >> 21) & 7
            v = pv & 0xFFFFF
            col = v & 127
            b16 = col & 112
            b = col & 15
            h64 = (mvec[j] & 1) * 64
            accs = []
            for k in range(_DIM // 16):
                ck = crow_v[gslot, j, pl.ds(h64 + 16 * k, 16)]
                acc = jnp.zeros((16,), jnp.float32)
                for l in range(16):
                    d = 16 * k + l
                    tv = tblk_v[tslot, d, pl.ds(b16, 16)]
                    acc = acc + _lane_perm(ck, lane * 0 + l) * tv
                accs.append(acc)
            acc = (accs[0] + accs[1]) + (accs[2] + accs[3])
            vec = jnp.where(lane == j, _lane_perm(acc, lane * 0 + b), vec)
            if j + _NBUF < 16:
                vn = idxt[j + _NBUF]
            else:
                vn = jnp.where(last, 0, idxt_n[j + _NBUF - 16])
            fire_t(vn)
        sim_v[pl.ds(goff, 16)] = vec
        return 0

    lax.fori_loop(0, _NGRP, body, 0)

    pltpu.sync_copy(sim_v, sim_sh.at[pl.ds(sid * _BPW, _BPW)])
    plsc.subcore_barrier()

    @pl.when(sid == 0)
    def _():
        pltpu.sync_copy(sim_sh, sim_hbm.at[pl.ds(cid * (_B // _NC),
                                                 _B // _NC)])


@functools.partial(
    pl.kernel,
    out_type=jax.ShapeDtypeStruct((_B,), jnp.float32),
    mesh=plsc.VectorSubcoreMesh(core_axis_name="c", subcore_axis_name="s"),
    compiler_params=pltpu.CompilerParams(use_tc_tiling_on_sc=True),
    scratch_types=[
        pltpu.VMEM((8, _BPW), jnp.int32),
        pltpu.VMEM((8, _BPW), jnp.int32),
        pltpu.VMEM((_NBUF, _DIM, 128), jnp.float32),
        pltpu.VMEM((2, 16, 128), jnp.float32),
        pltpu.VMEM((_BPW,), jnp.float32),
        pltpu.VMEM_SHARED((_B // _NC,), jnp.float32),
        pltpu.SemaphoreType.DMA((_NBUF,)),
        pltpu.SemaphoreType.DMA((2,)),
    ],
)
def _sc_simt(pt_hbm, m_hbm, wout_hbm, crows_hbm, sim_hbm,
             tidx_v, m_v, tblk_v, crow_v, sim_v, sim_sh, tsems, gsems):
    _sc_simt_kernel(pt_hbm, m_hbm, wout_hbm, crows_hbm, sim_hbm,
                    tidx_v, m_v, tblk_v, crow_v, sim_v, sim_sh,
                    tsems, gsems)


def _tc_bce_kernel(sim_ref, y_ref, out_ref):
    s = sim_ref[...]
    y = y_ref[...]
    t = jnp.maximum(s, 0.0) - s * y + jnp.log1p(jnp.exp(-jnp.abs(s)))
    out_ref[...] = (jnp.sum(t) * (1.0 / _B))[None, None]


def kernel(center_words, target_words, label, W_in, W_out):
    cw32 = center_words.astype(jnp.int32)
    tw32 = target_words.astype(jnp.int32)
    permc = jnp.argsort(cw32)
    permt = jnp.argsort(tw32)
    pc = _pack_sorted(cw32[permc].reshape(_NW, _BPW))
    pt = _pack_sorted(tw32[permt].reshape(_NW, _BPW))
    invc = jnp.zeros((_B,), jnp.int32).at[permc].set(
        jnp.arange(_B, dtype=jnp.int32))
    m = invc[permt].reshape(_NW, _BPW)
    yp = label[permt].astype(jnp.float32)
    crows = _sc_crows(pc, W_in.T)
    sim = _sc_simt(pt, m, W_out.T, crows)
    loss = pl.pallas_call(
        _tc_bce_kernel,
        out_shape=jax.ShapeDtypeStruct((1, 1), jnp.float32),
    )(sim.reshape(128, 128), yp.reshape(128, 128))
    return loss.reshape(())


# speedup vs baseline: 1.0972x; 1.0972x over previous
import functools

import jax
import jax.numpy as jnp
from jax import lax
from jax.experimental import pallas as pl
from jax.experimental.pallas import tpu as pltpu
from jax.experimental.pallas import tpu_sc as plsc

from jax._src import core as _jax_core
from jax._src.pallas.mosaic import sc_lowering as _scl
from jax.experimental.mosaic.dialects import tpu as _tpu_dialect

_lane_perm_p = _jax_core.Primitive("sc_lane_perm")


@_lane_perm_p.def_abstract_eval
def _lane_perm_abs(x, idx):
    return _jax_core.ShapedArray(x.shape, x.dtype)


@_scl.register_lowering_rule(_lane_perm_p)
def _lane_perm_lowering(ctx, x, idx):
    return _tpu_dialect.dynamic_gather(x, idx, dimensions=[0])


def _lane_perm(x, idx):
    return _lane_perm_p.bind(x, idx)


_VOCAB = 1000000
_DIM = 64
_B = 16384

_NC = 2
_NS = 16
_NW = _NC * _NS
_BPW = _B // _NW
_NGRP = _BPW // 16
_NBUF = 4
_NBUFC = 6


def _sc_sim_kernel(cw_hbm, tw_hbm, win_hbm, wout_hbm, sim_hbm,
                   cidx_v, tidx_v, cblk_v, tblk_v, sim_v, sim_sh,
                   csems, tsems):
    cid = lax.axis_index("c")
    sid = lax.axis_index("s")
    wid = cid * _NS + sid
    r = wid % 8

    slab = (wid // 8) * 8
    pltpu.sync_copy(cw_hbm.at[pl.ds(slab, 8)], cidx_v)
    pltpu.sync_copy(tw_hbm.at[pl.ds(slab, 8)], tidx_v)

    lane = lax.iota(jnp.int32, 16)

    def fire(v, tbl_hbm, blk_v, sems, slot):
        vb = pl.multiple_of(((v & 0xFFFFF) >> 7) * 128, 128)
        return pltpu.async_copy(
            tbl_hbm.at[pl.ds(0, _DIM), pl.ds(vb, 128)],
            blk_v.at[slot], sems.at[slot])

    def fire_c(pv):
        @pl.when(((pv >> 20) & 1) == 1)
        def _():
            fire(pv, win_hbm, cblk_v, csems, (pv >> 21) & 7)

    def wait(tbl_hbm, blk_v, sems, slot):
        pltpu.make_async_copy(
            tbl_hbm.at[pl.ds(0, _DIM), pl.ds(0, 128)],
            blk_v.at[slot], sems.at[slot]).wait()

    def wait_c(pv):
        @pl.when(((pv >> 20) & 1) == 1)
        def _():
            wait(win_hbm, cblk_v, csems, (pv >> 21) & 7)

    idxc0 = cidx_v[r, pl.ds(0, 16)]
    idxt0 = tidx_v[r, pl.ds(0, 16)]
    for j in range(_NBUFC):
        fire_c(idxc0[j])
    for j in range(_NBUF):
        fire(idxt0[j], wout_hbm, tblk_v, tsems, j)

    def body(g, _):
        goff = g * 16
        noff = jnp.minimum(goff + 16, _BPW - 16)
        idxc = cidx_v[r, pl.ds(goff, 16)]
        idxt = tidx_v[r, pl.ds(goff, 16)]
        idxc_n = cidx_v[r, pl.ds(noff, 16)]
        idxt_n = tidx_v[r, pl.ds(noff, 16)]
        last = g == _NGRP - 1
        vec = jnp.zeros((16,), jnp.float32)
        for j in range(16):
            slot = j % _NBUF
            pv = idxc[j]
            cslot = (pv >> 21) & 7
            wait_c(pv)
            wait(wout_hbm, tblk_v, tsems, slot)
            v_c = pv & 0xFFFFF
            v_t = idxt[j]
            col_c, col_t = v_c & 127, v_t & 127
            b16c, b16t = col_c & 112, col_t & 112
            a, b = col_c & 15, col_t & 15
            rot = (b - a) & 15
            pidx = (lane + rot) & 15
            acc = jnp.zeros((16,), jnp.float32)
            for d in range(_DIM):
                cv = cblk_v[cslot, d, pl.ds(b16c, 16)]
                tv = tblk_v[slot, d, pl.ds(b16t, 16)]
                acc = acc + cv * _lane_perm(tv, pidx)
            vec = jnp.where(lane == j, _lane_perm(acc, lane * 0 + a), vec)
            if j + _NBUFC < 16:
                vn_c = idxc[j + _NBUFC]
            else:
                vn_c = jnp.where(last, 0, idxc_n[j + _NBUFC - 16])
            if j + _NBUF < 16:
                vn_t = idxt[j + _NBUF]
            else:
                vn_t = jnp.where(last, idxt[15], idxt_n[j + _NBUF - 16])
            fire_c(vn_c)
            fire(vn_t, wout_hbm, tblk_v, tsems, slot)
        sim_v[pl.ds(goff, 16)] = vec
        return 0

    lax.fori_loop(0, _NGRP, body, 0)

    for j in range(_NBUF):
        wait(wout_hbm, tblk_v, tsems, j)

    pltpu.sync_copy(sim_v, sim_sh.at[pl.ds(sid * _BPW, _BPW)])
    plsc.subcore_barrier()

    @pl.when(sid == 0)
    def _():
        pltpu.sync_copy(sim_sh, sim_hbm.at[pl.ds(cid * (_B // _NC),
                                                 _B // _NC)])


@functools.partial(
    pl.kernel,
    out_type=jax.ShapeDtypeStruct((_B,), jnp.float32),
    mesh=plsc.VectorSubcoreMesh(core_axis_name="c", subcore_axis_name="s"),
    compiler_params=pltpu.CompilerParams(use_tc_tiling_on_sc=True),
    scratch_types=[
        pltpu.VMEM((8, _BPW), jnp.int32),
        pltpu.VMEM((8, _BPW), jnp.int32),
        pltpu.VMEM((_NBUFC, _DIM, 128), jnp.float32),
        pltpu.VMEM((_NBUF, _DIM, 128), jnp.float32),
        pltpu.VMEM((_BPW,), jnp.float32),
        pltpu.VMEM_SHARED((_B // _NC,), jnp.float32),
        pltpu.SemaphoreType.DMA((_NBUFC,)),
        pltpu.SemaphoreType.DMA((_NBUF,)),
    ],
)
def _sc_sim(cw_hbm, tw_hbm, win_hbm, wout_hbm, sim_hbm,
            cidx_v, tidx_v, cblk_v, tblk_v, sim_v, sim_sh, csems, tsems):
    _sc_sim_kernel(cw_hbm, tw_hbm, win_hbm, wout_hbm, sim_hbm,
                   cidx_v, tidx_v, cblk_v, tblk_v, sim_v, sim_sh,
                   csems, tsems)


def _tc_bce_kernel(sim_ref, y_ref, out_ref):
    s = sim_ref[...]
    y = y_ref[...]
    t = jnp.maximum(s, 0.0) - s * y + jnp.log1p(jnp.exp(-jnp.abs(s)))
    out_ref[...] = (jnp.sum(t) * (1.0 / _B))[None, None]


def kernel(center_words, target_words, label, W_in, W_out):
    cw32 = center_words.astype(jnp.int32)
    perm = jnp.argsort(cw32)
    sv = cw32[perm].reshape(_NW, _BPW)
    tw = target_words.astype(jnp.int32)[perm].reshape(_NW, _BPW)
    yp = label[perm].astype(jnp.float32)
    prevb = jnp.concatenate(
        [jnp.full((_NW, 1), -1, jnp.int32), sv[:, :-1] >> 7], axis=1)
    nb = ((sv >> 7) != prevb).astype(jnp.int32)
    uslot = (jnp.cumsum(nb, axis=1) - 1) % 6
    pc = sv | (nb << 20) | (uslot << 21)
    sim = _sc_sim(pc, tw, W_in.T, W_out.T)
    loss = pl.pallas_call(
        _tc_bce_kernel,
        out_shape=jax.ShapeDtypeStruct((1, 1), jnp.float32),
    )(sim.reshape(128, 128), yp.reshape(128, 128))
    return loss.reshape(())
